# Initial kernel scaffold; baseline (speedup 1.0000x reference)
import jax
import jax.numpy as jnp
from jax.experimental import pallas as pl


def kernel(x, W, b, codebook):
    raise NotImplementedError("write your pallas kernel here")


# speedup vs baseline: 1.0879x; 1.0879x over previous
import functools

import jax
import jax.numpy as jnp
from jax import lax
from jax.experimental import pallas as pl
from jax.experimental.pallas import tpu as pltpu
from jax.experimental.pallas import tpu_sc as plsc

DIM = 1024
NUM_VARS = 8192
VQ_DIM = 256
GAMMA = 0.25

TN = 2048
TV = 1024


def _proj_body(x_ref, W_ref, b_ref, z_ref):
    zt = lax.dot_general(W_ref[...], x_ref[...], (((0,), (1,)), ((), ())),
                         preferred_element_type=jnp.float32)
    z_ref[...] = zt.T + b_ref[...]


def _project(xf, W, b):
    n = xf.shape[0]
    return pl.pallas_call(
        _proj_body, grid=(n // TN,),
        in_specs=[pl.BlockSpec((TN, DIM), lambda i: (i, 0)),
                  pl.BlockSpec((DIM, VQ_DIM), lambda i: (0, 0)),
                  pl.BlockSpec((1, VQ_DIM), lambda i: (0, 0))],
        out_specs=pl.BlockSpec((TN, VQ_DIM), lambda i: (i, 0)),
        out_shape=jax.ShapeDtypeStruct((n, VQ_DIM), jnp.float32),
    )(xf, W, b.reshape(1, VQ_DIM))


def _argmin_body(z_ref, cb_ref, zsq_ref, k_ref, best_d, best_i):
    j = pl.program_id(1)
    nj = pl.num_programs(1)

    @pl.when(j == 0)
    def _():
        best_d[...] = jnp.full((TN, 1), jnp.inf, jnp.float32)
        best_i[...] = jnp.zeros((TN, 1), jnp.int32)

    cb = cb_ref[...]
    c_sq = jnp.sum(cb * cb, axis=1, keepdims=True)
    m = lax.dot_general(z_ref[...], cb, (((1,), (1,)), ((), ())),
                        preferred_element_type=jnp.float32)
    d2 = (zsq_ref[...] + c_sq.T) - 2.0 * m
    dist = jnp.sqrt(jnp.maximum(d2, 0.0))
    mn = jnp.min(dist, axis=1, keepdims=True)
    ids = lax.broadcasted_iota(jnp.int32, (TN, TV), 1)
    local = jnp.min(jnp.where(dist == mn, ids, NUM_VARS),
                    axis=1, keepdims=True) + j * TV
    better = mn < best_d[...]
    best_d[...] = jnp.where(better, mn, best_d[...])
    best_i[...] = jnp.where(better, local, best_i[...])

    @pl.when(j == nj - 1)
    def _():
        k_ref[...] = best_i[...]


def _argmin(z, zsq, codebook):
    n = z.shape[0]
    k = pl.pallas_call(
        _argmin_body, grid=(n // TN, NUM_VARS // TV),
        in_specs=[pl.BlockSpec((TN, VQ_DIM), lambda i, j: (i, 0)),
                  pl.BlockSpec((TV, VQ_DIM), lambda i, j: (j, 0)),
                  pl.BlockSpec((TN, 1), lambda i, j: (i, 0))],
        out_specs=pl.BlockSpec((TN, 1), lambda i, j: (i, 0)),
        out_shape=jax.ShapeDtypeStruct((n, 1), jnp.int32),
        scratch_shapes=[pltpu.VMEM((TN, 1), jnp.float32),
                        pltpu.VMEM((TN, 1), jnp.int32)],
    )(z, codebook, zsq.reshape(n, 1))
    return k.reshape(-1)


def _sc_gather(codebook, k):
    n = k.shape[0]
    info = plsc.get_sparse_core_info()
    nw = info.num_cores * info.num_subcores
    b_per_w = n // nw
    mesh = plsc.VectorSubcoreMesh(core_axis_name="c", subcore_axis_name="s")

    @functools.partial(
        pl.kernel, mesh=mesh,
        out_type=jax.ShapeDtypeStruct((n, VQ_DIM), jnp.float32),
        scratch_types=[pltpu.VMEM((b_per_w,), jnp.int32),
                       pltpu.VMEM((b_per_w, VQ_DIM), jnp.float32),
                       pltpu.SemaphoreType.DMA],
    )
    def gk(cb_hbm, k_hbm, out_hbm, idx_v, rows_v, sem):
        wid = lax.axis_index("s") * info.num_cores + lax.axis_index("c")
        base = wid * b_per_w
        pltpu.sync_copy(k_hbm.at[pl.ds(base, b_per_w)], idx_v)
        pltpu.async_copy(cb_hbm.at[idx_v], rows_v, sem).wait()
        pltpu.sync_copy(rows_v, out_hbm.at[pl.ds(base, b_per_w)])

    return gk(codebook, k)


_HB = 512


def _hist_body(k_ref, perp_ref, counts_ref):
    i = pl.program_id(0)
    ni = pl.num_programs(0)

    @pl.when(i == 0)
    def _():
        counts_ref[...] = jnp.zeros((1, NUM_VARS), jnp.float32)

    kb = k_ref[...]
    bins = lax.broadcasted_iota(jnp.int32, (_HB, NUM_VARS), 1)
    eqf = (kb == bins).astype(jnp.float32)
    counts_ref[...] += jnp.sum(eqf, axis=0, keepdims=True)

    @pl.when(i == ni - 1)
    def _():
        p = counts_ref[...] * (1.0 / 8192.0)
        ent = jnp.sum(p * jnp.log(p + 1e-07), axis=1, keepdims=True)
        perp_ref[...] = jnp.exp(-ent)


def _perplexity(k2d):
    n = k2d.shape[0]
    return pl.pallas_call(
        _hist_body, grid=(n // _HB,),
        in_specs=[pl.BlockSpec((_HB, 1), lambda i: (i, 0))],
        out_specs=pl.BlockSpec((1, 1), lambda i: (0, 0)),
        out_shape=jax.ShapeDtypeStruct((1, 1), jnp.float32),
        scratch_shapes=[pltpu.VMEM((1, NUM_VARS), jnp.float32)],
    )(k2d)


def _st_body(z_ref, q_ref, qst_ref, loss_ref, acc_ref):
    i = pl.program_id(0)
    ni = pl.num_programs(0)

    @pl.when(i == 0)
    def _():
        acc_ref[...] = jnp.zeros((1, 1), jnp.float32)

    z = z_ref[...]
    q = q_ref[...]
    diff = q - z
    qst_ref[...] = z + diff
    acc_ref[...] += jnp.sum(diff * diff).reshape(1, 1)

    @pl.when(i == ni - 1)
    def _():
        mse = acc_ref[...] * (1.0 / (8192.0 * 256.0))
        loss_ref[...] = (GAMMA + 1.0) * mse


def _st_loss(z, q):
    n = z.shape[0]
    return pl.pallas_call(
        _st_body, grid=(n // TN,),
        in_specs=[pl.BlockSpec((TN, VQ_DIM), lambda i: (i, 0)),
                  pl.BlockSpec((TN, VQ_DIM), lambda i: (i, 0))],
        out_specs=[pl.BlockSpec((TN, VQ_DIM), lambda i: (i, 0)),
                   pl.BlockSpec((1, 1), lambda i: (0, 0))],
        out_shape=[jax.ShapeDtypeStruct((n, VQ_DIM), jnp.float32),
                   jax.ShapeDtypeStruct((1, 1), jnp.float32)],
        scratch_shapes=[pltpu.VMEM((1, 1), jnp.float32)],
    )(z, q)


def kernel(x, W, b, codebook):
    bsz, tsz, fsz = x.shape
    xf = x.reshape(-1, fsz)
    z = _project(xf, W, b)
    zsq = jnp.sum(z * z, axis=1)
    k = _argmin(z, zsq, codebook)
    quantized = _sc_gather(codebook, k)
    perp = _perplexity(k.reshape(-1, 1))
    qst, loss = _st_loss(z, quantized)
    code_perplexity = perp.reshape(())
    latent_loss = loss.reshape(())
    quantized_out = jnp.transpose(qst.reshape(bsz, tsz, -1), (0, 2, 1))
    return (code_perplexity, latent_loss, k, quantized_out)
